# Initial kernel scaffold; baseline (speedup 1.0000x reference)
import functools

import jax
import jax.numpy as jnp
from jax import lax
from jax.experimental import pallas as pl
from jax.experimental.pallas import tpu as pltpu
from jax.experimental.pallas import tpu_sc as plsc

N = 10000
E = 320000
F_IN = 128
HID = 64
OUT = 300

NC = 2
NS = 16
NW = NC * NS
EPW = E // NW
CHUNK = 80
NCHUNK = EPW // CHUNK
RPT = N // NS
ZR = 125
DEGW = 16


def _proj_body(x_ref, wlt_ref, wrt_ref, xl_ref, xr_ref):
    x = x_ref[...]
    xl_ref[...] = jnp.dot(x, wlt_ref[...], preferred_element_type=jnp.float32)
    xr_ref[...] = jnp.dot(x, wrt_ref[...], preferred_element_type=jnp.float32)


def _proj(x, wlt, wrt):
    rb = 1000
    return pl.pallas_call(
        _proj_body,
        grid=(N // rb,),
        in_specs=[
            pl.BlockSpec((rb, F_IN), lambda i: (i, 0)),
            pl.BlockSpec((F_IN, HID), lambda i: (0, 0)),
            pl.BlockSpec((F_IN, HID), lambda i: (0, 0)),
        ],
        out_specs=[
            pl.BlockSpec((rb, HID), lambda i: (i, 0)),
            pl.BlockSpec((rb, HID), lambda i: (i, 0)),
        ],
        out_shape=[
            jax.ShapeDtypeStruct((N, HID), jnp.float32),
            jax.ShapeDtypeStruct((N, HID), jnp.float32),
        ],
    )(x, wlt, wrt)


def _sc_aggregate(xl, src, dst):
    mesh = plsc.VectorSubcoreMesh(core_axis_name="c", subcore_axis_name="s")

    @functools.partial(
        pl.kernel,
        mesh=mesh,
        out_type=[
            jax.ShapeDtypeStruct((NC * N, HID), jnp.float32),
            jax.ShapeDtypeStruct((NC * N, DEGW), jnp.float32),
        ],
        scratch_types=[
            pltpu.VMEM_SHARED((N, HID), jnp.float32),
            pltpu.VMEM_SHARED((N, DEGW), jnp.float32),
            pltpu.VMEM((CHUNK,), jnp.int32),
            pltpu.VMEM((CHUNK,), jnp.int32),
            pltpu.VMEM((CHUNK, HID), jnp.float32),
            pltpu.VMEM((CHUNK, DEGW), jnp.float32),
            pltpu.VMEM((ZR, HID), jnp.float32),
            pltpu.VMEM((ZR, DEGW), jnp.float32),
            pltpu.SemaphoreType.DMA,
        ],
    )
    def sc_kernel(xl_hbm, src_hbm, dst_hbm, agg_out, deg_out,
                  agg_sh, deg_sh, src_v, dst_v, rows_v, ones_v,
                  zrow_v, zdeg_v, sem):
        cid = lax.axis_index("c")
        sid = lax.axis_index("s")

        z16 = jnp.zeros((16,), jnp.float32)
        o16 = jnp.ones((16,), jnp.float32)

        def fill_zrow(i, carry):
            for j in range(HID // 16):
                zrow_v[i, pl.ds(j * 16, 16)] = z16
            zdeg_v[i, :] = z16
            return carry

        lax.fori_loop(0, ZR, fill_zrow, 0)

        def fill_ones(i, carry):
            ones_v[i, :] = o16
            return carry

        lax.fori_loop(0, CHUNK, fill_ones, 0)

        rbase = sid * RPT
        for k in range(RPT // ZR):
            pltpu.sync_copy(zrow_v, agg_sh.at[pl.ds(rbase + k * ZR, ZR)])
            pltpu.sync_copy(zdeg_v, deg_sh.at[pl.ds(rbase + k * ZR, ZR)])
        plsc.subcore_barrier()

        wid = sid * NC + cid
        ebase = wid * EPW

        def chunk_body(i, carry):
            off = ebase + i * CHUNK
            pltpu.sync_copy(src_hbm.at[pl.ds(off, CHUNK)], src_v)
            pltpu.sync_copy(dst_hbm.at[pl.ds(off, CHUNK)], dst_v)
            pltpu.async_copy(xl_hbm.at[src_v], rows_v, sem).wait()
            pltpu.sync_copy(rows_v, agg_sh.at[dst_v], add=True)
            pltpu.sync_copy(ones_v, deg_sh.at[dst_v], add=True)
            return carry

        lax.fori_loop(0, NCHUNK, chunk_body, 0)
        plsc.subcore_barrier()

        obase = cid * N + rbase
        pltpu.sync_copy(agg_sh.at[pl.ds(rbase, RPT)],
                        agg_out.at[pl.ds(obase, RPT)])
        pltpu.sync_copy(deg_sh.at[pl.ds(rbase, RPT)],
                        deg_out.at[pl.ds(obase, RPT)])

    return sc_kernel(xl, src, dst)


def _head_body(agg_ref, deg_ref, xr_ref, bl_ref, w2t_ref, b2_ref, y_ref):
    agg = agg_ref[0] + agg_ref[1]
    deg = jnp.maximum(jnp.sum(deg_ref[...], axis=(0, 2)), 1.0)
    h = agg / deg[:, None] + bl_ref[...] + xr_ref[...]
    h = jnp.maximum(h, 0.0)
    y_ref[...] = (jnp.dot(h, w2t_ref[...], preferred_element_type=jnp.float32)
                  + b2_ref[...])


def _head(agg_parts, deg_parts, xr, bl, w2t, b2):
    rb = 1000
    return pl.pallas_call(
        _head_body,
        grid=(N // rb,),
        in_specs=[
            pl.BlockSpec((NC, rb, HID), lambda i: (0, i, 0)),
            pl.BlockSpec((NC, rb, DEGW), lambda i: (0, i, 0)),
            pl.BlockSpec((rb, HID), lambda i: (i, 0)),
            pl.BlockSpec((1, HID), lambda i: (0, 0)),
            pl.BlockSpec((HID, OUT), lambda i: (0, 0)),
            pl.BlockSpec((1, OUT), lambda i: (0, 0)),
        ],
        out_specs=pl.BlockSpec((rb, OUT), lambda i: (i, 0)),
        out_shape=jax.ShapeDtypeStruct((N, OUT), jnp.float32),
    )(agg_parts, deg_parts, xr, bl, w2t, b2)


def kernel(x, edge_index, batch, Wl, bl, Wr, W2, b2):
    src = edge_index[0]
    dst = edge_index[1]
    xl, xr = _proj(x, Wl.T, Wr.T)
    agg_flat, deg_flat = _sc_aggregate(xl, src, dst)
    agg_parts = agg_flat.reshape(NC, N, HID)
    deg_parts = deg_flat.reshape(NC, N, DEGW)
    y = _head(agg_parts, deg_parts, xr, bl.reshape(1, HID), W2.T,
              b2.reshape(1, OUT))
    return y.reshape(-1, 100)


# speedup vs baseline: 6.0826x; 6.0826x over previous
import functools

import jax
import jax.numpy as jnp
from jax import lax
from jax.experimental import pallas as pl
from jax.experimental.pallas import tpu as pltpu
from jax.experimental.pallas import tpu_sc as plsc

N = 10000
E = 320000
F_IN = 128
HID = 64
OUT = 300

NC = 2
NS = 16
NW = NC * NS
EPW = E // NW
CHUNK = 80
NCHUNK = EPW // CHUNK
NP = 10240
RPT = NP // NS
ZR = 128
DEGW = 16


def _proj_body(x_ref, wlt_ref, wrt_ref, xl_ref, xr_ref):
    x = x_ref[...]
    xl_ref[...] = jnp.dot(x, wlt_ref[...], preferred_element_type=jnp.float32)
    xr_ref[...] = jnp.dot(x, wrt_ref[...], preferred_element_type=jnp.float32)


def _proj(x, wlt, wrt):
    rb = 1000
    return pl.pallas_call(
        _proj_body,
        grid=(N // rb,),
        in_specs=[
            pl.BlockSpec((rb, F_IN), lambda i: (i, 0)),
            pl.BlockSpec((F_IN, HID), lambda i: (0, 0)),
            pl.BlockSpec((F_IN, HID), lambda i: (0, 0)),
        ],
        out_specs=[
            pl.BlockSpec((rb, HID), lambda i: (i, 0)),
            pl.BlockSpec((rb, HID), lambda i: (i, 0)),
        ],
        out_shape=[
            jax.ShapeDtypeStruct((N, HID), jnp.float32),
            jax.ShapeDtypeStruct((N, HID), jnp.float32),
        ],
    )(x, wlt, wrt)


def _sc_aggregate(xl, src, dst):
    mesh = plsc.VectorSubcoreMesh(core_axis_name="c", subcore_axis_name="s")

    @functools.partial(
        pl.kernel,
        mesh=mesh,
        out_type=[
            jax.ShapeDtypeStruct((NC * NP, HID), jnp.float32),
            jax.ShapeDtypeStruct((NC * NP, DEGW), jnp.float32),
        ],
        scratch_types=[
            pltpu.VMEM_SHARED((NP, HID), jnp.float32),
            pltpu.VMEM_SHARED((NP, DEGW), jnp.float32),
            pltpu.VMEM((CHUNK,), jnp.int32),
            pltpu.VMEM((CHUNK,), jnp.int32),
            pltpu.VMEM((CHUNK, HID), jnp.float32),
            pltpu.VMEM((CHUNK, DEGW), jnp.float32),
            pltpu.VMEM((ZR, HID), jnp.float32),
            pltpu.VMEM((ZR, DEGW), jnp.float32),
            pltpu.SemaphoreType.DMA,
        ],
        compiler_params=pltpu.CompilerParams(use_tc_tiling_on_sc=False),
    )
    def sc_kernel(xl_hbm, src_hbm, dst_hbm, agg_out, deg_out,
                  agg_sh, deg_sh, src_v, dst_v, rows_v, ones_v,
                  zrow_v, zdeg_v, sem):
        cid = lax.axis_index("c")
        sid = lax.axis_index("s")

        z16 = jnp.zeros((16,), jnp.float32)
        o16 = jnp.ones((16,), jnp.float32)

        def fill_zrow(i, carry):
            for j in range(HID // 16):
                zrow_v[i, pl.ds(j * 16, 16)] = z16
            zdeg_v[i, :] = z16
            return carry

        lax.fori_loop(0, ZR, fill_zrow, 0)

        def fill_ones(i, carry):
            ones_v[i, :] = o16
            return carry

        lax.fori_loop(0, CHUNK, fill_ones, 0)

        rbase = sid * RPT
        for k in range(RPT // ZR):
            pltpu.sync_copy(zrow_v, agg_sh.at[pl.ds(rbase + k * ZR, ZR)])
            pltpu.sync_copy(zdeg_v, deg_sh.at[pl.ds(rbase + k * ZR, ZR)])
        plsc.subcore_barrier()

        wid = sid * NC + cid
        ebase = wid * EPW

        def chunk_body(i, carry):
            off = ebase + i * CHUNK
            pltpu.sync_copy(src_hbm.at[pl.ds(off, CHUNK)], src_v)
            pltpu.sync_copy(dst_hbm.at[pl.ds(off, CHUNK)], dst_v)
            pltpu.async_copy(xl_hbm.at[src_v], rows_v, sem).wait()
            pltpu.sync_copy(rows_v, agg_sh.at[dst_v], add=True)
            pltpu.sync_copy(ones_v, deg_sh.at[dst_v], add=True)
            return carry

        lax.fori_loop(0, NCHUNK, chunk_body, 0)
        plsc.subcore_barrier()

        obase = cid * NP + rbase
        pltpu.sync_copy(agg_sh.at[pl.ds(rbase, RPT)],
                        agg_out.at[pl.ds(obase, RPT)])
        pltpu.sync_copy(deg_sh.at[pl.ds(rbase, RPT)],
                        deg_out.at[pl.ds(obase, RPT)])

    return sc_kernel(xl, src, dst)


def _head_body(agg_ref, deg_ref, xr_ref, bl_ref, w2t_ref, b2_ref, y_ref):
    agg = agg_ref[0] + agg_ref[1]
    deg = jnp.maximum(jnp.sum(deg_ref[...], axis=(0, 2)) * (1.0 / DEGW), 1.0)
    h = agg / deg[:, None] + bl_ref[...] + xr_ref[...]
    h = jnp.maximum(h, 0.0)
    y_ref[...] = (jnp.dot(h, w2t_ref[...], preferred_element_type=jnp.float32)
                  + b2_ref[...])


def _head(agg_parts, deg_parts, xr, bl, w2t, b2):
    rb = 1000
    return pl.pallas_call(
        _head_body,
        grid=(N // rb,),
        in_specs=[
            pl.BlockSpec((NC, rb, HID), lambda i: (0, i, 0)),
            pl.BlockSpec((NC, rb, DEGW), lambda i: (0, i, 0)),
            pl.BlockSpec((rb, HID), lambda i: (i, 0)),
            pl.BlockSpec((1, HID), lambda i: (0, 0)),
            pl.BlockSpec((HID, OUT), lambda i: (0, 0)),
            pl.BlockSpec((1, OUT), lambda i: (0, 0)),
        ],
        out_specs=pl.BlockSpec((rb, OUT), lambda i: (i, 0)),
        out_shape=jax.ShapeDtypeStruct((N, OUT), jnp.float32),
    )(agg_parts, deg_parts, xr, bl, w2t, b2)


def kernel(x, edge_index, batch, Wl, bl, Wr, W2, b2):
    src = edge_index[0]
    dst = edge_index[1]
    xl, xr = _proj(x, Wl.T, Wr.T)
    agg_flat, deg_flat = _sc_aggregate(xl, src, dst)
    agg_parts = agg_flat.reshape(NC, NP, HID)[:, :N]
    deg_parts = deg_flat.reshape(NC, NP, DEGW)[:, :N]
    y = _head(agg_parts, deg_parts, xr, bl.reshape(1, HID), W2.T,
              b2.reshape(1, OUT))
    return y.reshape(-1, 100)


# speedup vs baseline: 13.3999x; 2.2030x over previous
import functools

import jax
import jax.numpy as jnp
from jax import lax
from jax.experimental import pallas as pl
from jax.experimental.pallas import tpu as pltpu
from jax.experimental.pallas import tpu_sc as plsc

N = 10000
E = 320000
F_IN = 128
HID = 64
OUT = 300

NC = 2
NS = 16
NW = NC * NS
EPW = E // NW
CHUNK = 80
NCHUNK = EPW // CHUNK
NP = 10240
RPT = NP // NS
ZR = 128
DEGW = 16


def _proj_body(x_ref, wlt_ref, wrt_ref, xl_ref, xr_ref):
    x = x_ref[...]
    xl_ref[...] = jnp.dot(x, wlt_ref[...], preferred_element_type=jnp.float32)
    xr_ref[...] = jnp.dot(x, wrt_ref[...], preferred_element_type=jnp.float32)


def _proj(x, wlt, wrt):
    rb = 1000
    return pl.pallas_call(
        _proj_body,
        grid=(N // rb,),
        in_specs=[
            pl.BlockSpec((rb, F_IN), lambda i: (i, 0)),
            pl.BlockSpec((F_IN, HID), lambda i: (0, 0)),
            pl.BlockSpec((F_IN, HID), lambda i: (0, 0)),
        ],
        out_specs=[
            pl.BlockSpec((rb, HID), lambda i: (i, 0)),
            pl.BlockSpec((rb, HID), lambda i: (i, 0)),
        ],
        out_shape=[
            jax.ShapeDtypeStruct((N, HID), jnp.float32),
            jax.ShapeDtypeStruct((N, HID), jnp.float32),
        ],
    )(x, wlt, wrt)


NSLOT = 5
ROUNDS = NCHUNK // NSLOT
LRPT = N - (NS - 1) * RPT


def _sc_aggregate(xl, src3, dst3):
    mesh = plsc.VectorSubcoreMesh(core_axis_name="c", subcore_axis_name="s")

    @functools.partial(
        pl.kernel,
        mesh=mesh,
        out_type=[
            jax.ShapeDtypeStruct((NC * N, HID), jnp.float32),
            jax.ShapeDtypeStruct((NC * N, DEGW), jnp.float32),
        ],
        scratch_types=[
            pltpu.VMEM_SHARED((NP, HID), jnp.float32),
            pltpu.VMEM_SHARED((NP, DEGW), jnp.float32),
            pltpu.VMEM((NCHUNK, CHUNK), jnp.int32),
            pltpu.VMEM((NCHUNK, CHUNK), jnp.int32),
            pltpu.VMEM((NSLOT, 2, CHUNK, HID), jnp.float32),
            pltpu.VMEM((CHUNK, DEGW), jnp.float32),
            pltpu.VMEM((CHUNK, DEGW), jnp.float32),
            pltpu.SemaphoreType.DMA((NSLOT, 2)),
            pltpu.SemaphoreType.DMA((NSLOT, 2)),
            pltpu.SemaphoreType.DMA((NSLOT, 2)),
        ],
        compiler_params=pltpu.CompilerParams(use_tc_tiling_on_sc=False),
    )
    def sc_kernel(xl_hbm, src_hbm, dst_hbm, agg_out, deg_out,
                  agg_sh, deg_sh, src_all, dst_all, rows_v, ones_v,
                  zdeg_v, sem_g, sem_s, sem_d):
        cid = lax.axis_index("c")
        sid = lax.axis_index("s")

        z16 = jnp.zeros((16,), jnp.float32)
        o16 = jnp.ones((16,), jnp.float32)

        zrow_v = rows_v.at[0, 0]

        def fill_bufs(i, carry):
            for j in range(HID // 16):
                zrow_v[i, pl.ds(j * 16, 16)] = z16
            zdeg_v[i, :] = z16
            ones_v[i, :] = o16
            return carry

        lax.fori_loop(0, CHUNK, fill_bufs, 0)

        rbase = sid * RPT
        for k in range(RPT // CHUNK):
            pltpu.sync_copy(zrow_v, agg_sh.at[pl.ds(rbase + k * CHUNK, CHUNK)])
            pltpu.sync_copy(zdeg_v, deg_sh.at[pl.ds(rbase + k * CHUNK, CHUNK)])
        wid = sid * NC + cid
        pltpu.sync_copy(src_hbm.at[wid], src_all)
        pltpu.sync_copy(dst_hbm.at[wid], dst_all)
        plsc.subcore_barrier()

        def gather(c, k, b):
            pltpu.async_copy(xl_hbm.at[src_all.at[c]], rows_v.at[k, b],
                             sem_g.at[k, b])

        def gather_wait(c, k, b):
            pltpu.make_async_copy(xl_hbm.at[src_all.at[c]], rows_v.at[k, b],
                                  sem_g.at[k, b]).wait()

        def scat(c, k, b):
            pltpu.async_copy(rows_v.at[k, b], agg_sh.at[dst_all.at[c]],
                             sem_s.at[k, b], add=True)
            pltpu.async_copy(ones_v, deg_sh.at[dst_all.at[c]],
                             sem_d.at[k, b], add=True)

        def scat_wait(c, k, b):
            pltpu.make_async_copy(rows_v.at[k, b], agg_sh.at[dst_all.at[c]],
                                  sem_s.at[k, b]).wait()
            pltpu.make_async_copy(ones_v, deg_sh.at[dst_all.at[c]],
                                  sem_d.at[k, b]).wait()

        for k in range(NSLOT):
            gather(k, k, 0)

        def round_body(r, b):
            nb = 1 - b
            for k in range(NSLOT):
                c = r * NSLOT + k
                pc = c + NSLOT

                @pl.when(pc < NCHUNK)
                def _():
                    @pl.when(r > 0)
                    def _():
                        scat_wait(c - NSLOT, k, nb)
                    gather(pc, k, nb)

                gather_wait(c, k, b)
                scat(c, k, b)

        def two_rounds(j2, carry):
            round_body(2 * j2, 0)
            round_body(2 * j2 + 1, 1)
            return carry

        lax.fori_loop(0, (ROUNDS - 1) // 2, two_rounds, 0)

        for k in range(NSLOT):
            c = (ROUNDS - 1) * NSLOT + k
            gather_wait(c, k, 0)
            scat(c, k, 0)
        for k in range(NSLOT):
            scat_wait((ROUNDS - 2) * NSLOT + k, k, 1)
            scat_wait((ROUNDS - 1) * NSLOT + k, k, 0)
        plsc.subcore_barrier()

        obase = cid * N + rbase

        @pl.when(sid < NS - 1)
        def _():
            pltpu.sync_copy(agg_sh.at[pl.ds(rbase, RPT)],
                            agg_out.at[pl.ds(obase, RPT)])
            pltpu.sync_copy(deg_sh.at[pl.ds(rbase, RPT)],
                            deg_out.at[pl.ds(obase, RPT)])

        @pl.when(sid == NS - 1)
        def _():
            pltpu.sync_copy(agg_sh.at[pl.ds(rbase, LRPT)],
                            agg_out.at[pl.ds(obase, LRPT)])
            pltpu.sync_copy(deg_sh.at[pl.ds(rbase, LRPT)],
                            deg_out.at[pl.ds(obase, LRPT)])

    return sc_kernel(xl, src3, dst3)


def _head_body(agg_ref, deg_ref, xr_ref, bl_ref, w2t_ref, b2_ref, y_ref):
    agg = agg_ref[0] + agg_ref[1]
    deg = jnp.maximum(jnp.sum(deg_ref[...], axis=(0, 2)) * (1.0 / DEGW), 1.0)
    h = agg / deg[:, None] + bl_ref[...] + xr_ref[...]
    h = jnp.maximum(h, 0.0)
    y_ref[...] = (jnp.dot(h, w2t_ref[...], preferred_element_type=jnp.float32)
                  + b2_ref[...])


def _head(agg_parts, deg_parts, xr, bl, w2t, b2):
    rb = 1000
    return pl.pallas_call(
        _head_body,
        grid=(N // rb,),
        in_specs=[
            pl.BlockSpec((NC, rb, HID), lambda i: (0, i, 0)),
            pl.BlockSpec((NC, rb, DEGW), lambda i: (0, i, 0)),
            pl.BlockSpec((rb, HID), lambda i: (i, 0)),
            pl.BlockSpec((1, HID), lambda i: (0, 0)),
            pl.BlockSpec((HID, OUT), lambda i: (0, 0)),
            pl.BlockSpec((1, OUT), lambda i: (0, 0)),
        ],
        out_specs=pl.BlockSpec((rb, OUT), lambda i: (i, 0)),
        out_shape=jax.ShapeDtypeStruct((N, OUT), jnp.float32),
    )(agg_parts, deg_parts, xr, bl, w2t, b2)


def kernel(x, edge_index, batch, Wl, bl, Wr, W2, b2):
    src3 = edge_index[0].reshape(NW, NCHUNK, CHUNK)
    dst3 = edge_index[1].reshape(NW, NCHUNK, CHUNK)
    xl, xr = _proj(x, Wl.T, Wr.T)
    agg_flat, deg_flat = _sc_aggregate(xl, src3, dst3)
    agg_parts = agg_flat.reshape(NC, N, HID)
    deg_parts = deg_flat.reshape(NC, N, DEGW)
    y = _head(agg_parts, deg_parts, xr, bl.reshape(1, HID), W2.T,
              b2.reshape(1, OUT))
    return y.reshape(-1, 100)


# speedup vs baseline: 14.2299x; 1.0619x over previous
import functools

import jax
import jax.numpy as jnp
from jax import lax
from jax.experimental import pallas as pl
from jax.experimental.pallas import tpu as pltpu
from jax.experimental.pallas import tpu_sc as plsc

N = 10000
E = 320000
F_IN = 128
HID = 64
OUT = 300

NC = 2
NS = 16
NW = NC * NS
EPW = E // NW
CHUNK = 80
NCHUNK = EPW // CHUNK
NP = 10240
RPT = NP // NS
ZR = 128
DEGW = 16


def _proj_body(x_ref, wlt_ref, wrt_ref, xl_ref, xr_ref):
    x = x_ref[...]
    xl_ref[...] = jnp.dot(x, wlt_ref[...], preferred_element_type=jnp.float32)
    xr_ref[...] = jnp.dot(x, wrt_ref[...], preferred_element_type=jnp.float32)


def _proj(x, wlt, wrt):
    rb = 1000
    return pl.pallas_call(
        _proj_body,
        grid=(N // rb,),
        in_specs=[
            pl.BlockSpec((rb, F_IN), lambda i: (i, 0)),
            pl.BlockSpec((F_IN, HID), lambda i: (0, 0)),
            pl.BlockSpec((F_IN, HID), lambda i: (0, 0)),
        ],
        out_specs=[
            pl.BlockSpec((rb, HID), lambda i: (i, 0)),
            pl.BlockSpec((rb, HID), lambda i: (i, 0)),
        ],
        out_shape=[
            jax.ShapeDtypeStruct((N, HID), jnp.float32),
            jax.ShapeDtypeStruct((N, HID), jnp.float32),
        ],
    )(x, wlt, wrt)


NSLOT = 5
ROUNDS = NCHUNK // NSLOT
LRPT = N - (NS - 1) * RPT


def _sc_aggregate(xl, ei4):
    mesh = plsc.VectorSubcoreMesh(core_axis_name="c", subcore_axis_name="s")

    @functools.partial(
        pl.kernel,
        mesh=mesh,
        out_type=[
            jax.ShapeDtypeStruct((NC * N, HID), jnp.float32),
            jax.ShapeDtypeStruct((NC * N, DEGW), jnp.float32),
        ],
        scratch_types=[
            pltpu.VMEM_SHARED((NP, HID), jnp.float32),
            pltpu.VMEM_SHARED((NP, DEGW), jnp.float32),
            pltpu.VMEM((NCHUNK, CHUNK), jnp.int32),
            pltpu.VMEM((NCHUNK, CHUNK), jnp.int32),
            pltpu.VMEM((NSLOT, 2, CHUNK, HID), jnp.float32),
            pltpu.VMEM((CHUNK, DEGW), jnp.float32),
            pltpu.VMEM((CHUNK, DEGW), jnp.float32),
            pltpu.SemaphoreType.DMA((NSLOT, 2)),
            pltpu.SemaphoreType.DMA((NSLOT, 2)),
            pltpu.SemaphoreType.DMA((NSLOT, 2)),
        ],
        compiler_params=pltpu.CompilerParams(use_tc_tiling_on_sc=False),
    )
    def sc_kernel(xl_hbm, ei_hbm, agg_out, deg_out,
                  agg_sh, deg_sh, src_all, dst_all, rows_v, ones_v,
                  zdeg_v, sem_g, sem_s, sem_d):
        cid = lax.axis_index("c")
        sid = lax.axis_index("s")

        z16 = jnp.zeros((16,), jnp.float32)
        o16 = jnp.ones((16,), jnp.float32)

        zrow_v = rows_v.at[0, 0]

        def fill_bufs(i, carry):
            for j in range(HID // 16):
                zrow_v[i, pl.ds(j * 16, 16)] = z16
            zdeg_v[i, :] = z16
            ones_v[i, :] = o16
            return carry

        lax.fori_loop(0, CHUNK, fill_bufs, 0)

        rbase = sid * RPT
        for k in range(RPT // CHUNK):
            pltpu.sync_copy(zrow_v, agg_sh.at[pl.ds(rbase + k * CHUNK, CHUNK)])
            pltpu.sync_copy(zdeg_v, deg_sh.at[pl.ds(rbase + k * CHUNK, CHUNK)])
        wid = sid * NC + cid
        pltpu.sync_copy(ei_hbm.at[0, wid], src_all)
        pltpu.sync_copy(ei_hbm.at[1, wid], dst_all)
        plsc.subcore_barrier()

        def gather(c, k, b):
            pltpu.async_copy(xl_hbm.at[src_all.at[c]], rows_v.at[k, b],
                             sem_g.at[k, b])

        def gather_wait(c, k, b):
            pltpu.make_async_copy(xl_hbm.at[src_all.at[c]], rows_v.at[k, b],
                                  sem_g.at[k, b]).wait()

        def scat(c, k, b):
            pltpu.async_copy(rows_v.at[k, b], agg_sh.at[dst_all.at[c]],
                             sem_s.at[k, b], add=True)
            pltpu.async_copy(ones_v, deg_sh.at[dst_all.at[c]],
                             sem_d.at[k, b], add=True)

        def scat_wait(c, k, b):
            pltpu.make_async_copy(rows_v.at[k, b], agg_sh.at[dst_all.at[c]],
                                  sem_s.at[k, b]).wait()
            pltpu.make_async_copy(ones_v, deg_sh.at[dst_all.at[c]],
                                  sem_d.at[k, b]).wait()

        for k in range(NSLOT):
            gather(k, k, 0)

        def round_body(r, b):
            nb = 1 - b
            for k in range(NSLOT):
                c = r * NSLOT + k
                pc = c + NSLOT

                @pl.when(pc < NCHUNK)
                def _():
                    @pl.when(r > 0)
                    def _():
                        scat_wait(c - NSLOT, k, nb)
                    gather(pc, k, nb)

                gather_wait(c, k, b)
                scat(c, k, b)

        def two_rounds(j2, carry):
            round_body(2 * j2, 0)
            round_body(2 * j2 + 1, 1)
            return carry

        lax.fori_loop(0, (ROUNDS - 1) // 2, two_rounds, 0)

        for k in range(NSLOT):
            c = (ROUNDS - 1) * NSLOT + k
            gather_wait(c, k, 0)
            scat(c, k, 0)
        for k in range(NSLOT):
            scat_wait((ROUNDS - 2) * NSLOT + k, k, 1)
            scat_wait((ROUNDS - 1) * NSLOT + k, k, 0)
        plsc.subcore_barrier()

        obase = cid * N + rbase

        @pl.when(sid < NS - 1)
        def _():
            pltpu.sync_copy(agg_sh.at[pl.ds(rbase, RPT)],
                            agg_out.at[pl.ds(obase, RPT)])
            pltpu.sync_copy(deg_sh.at[pl.ds(rbase, RPT)],
                            deg_out.at[pl.ds(obase, RPT)])

        @pl.when(sid == NS - 1)
        def _():
            pltpu.sync_copy(agg_sh.at[pl.ds(rbase, LRPT)],
                            agg_out.at[pl.ds(obase, LRPT)])
            pltpu.sync_copy(deg_sh.at[pl.ds(rbase, LRPT)],
                            deg_out.at[pl.ds(obase, LRPT)])

    return sc_kernel(xl, ei4)


def _head_body(agg_ref, deg_ref, xr_ref, bl_ref, w2t_ref, b2_ref, y_ref):
    agg = agg_ref[0] + agg_ref[1]
    deg = jnp.maximum(jnp.sum(deg_ref[...], axis=(0, 2)) * (1.0 / DEGW), 1.0)
    h = agg / deg[:, None] + bl_ref[...] + xr_ref[...]
    h = jnp.maximum(h, 0.0)
    y_ref[...] = (jnp.dot(h, w2t_ref[...], preferred_element_type=jnp.float32)
                  + b2_ref[...])


def _head(agg_parts, deg_parts, xr, bl, w2t, b2):
    rb = 1000
    return pl.pallas_call(
        _head_body,
        grid=(N // rb,),
        in_specs=[
            pl.BlockSpec((NC, rb, HID), lambda i: (0, i, 0)),
            pl.BlockSpec((NC, rb, DEGW), lambda i: (0, i, 0)),
            pl.BlockSpec((rb, HID), lambda i: (i, 0)),
            pl.BlockSpec((1, HID), lambda i: (0, 0)),
            pl.BlockSpec((HID, OUT), lambda i: (0, 0)),
            pl.BlockSpec((1, OUT), lambda i: (0, 0)),
        ],
        out_specs=pl.BlockSpec((rb, OUT), lambda i: (i, 0)),
        out_shape=jax.ShapeDtypeStruct((N, OUT), jnp.float32),
    )(agg_parts, deg_parts, xr, bl, w2t, b2)


def kernel(x, edge_index, batch, Wl, bl, Wr, W2, b2):
    ei4 = edge_index.reshape(2, NW, NCHUNK, CHUNK)
    xl, xr = _proj(x, Wl.T, Wr.T)
    agg_flat, deg_flat = _sc_aggregate(xl, ei4)
    agg_parts = agg_flat.reshape(NC, N, HID)
    deg_parts = deg_flat.reshape(NC, N, DEGW)
    y = _head(agg_parts, deg_parts, xr, bl.reshape(1, HID), W2.T,
              b2.reshape(1, OUT))
    return y.reshape(-1, 100)
